# Initial kernel scaffold; baseline (speedup 1.0000x reference)
import functools

import jax
import jax.numpy as jnp
from jax import lax
from jax.experimental import pallas as pl
from jax.experimental.pallas import tpu as pltpu
from jax.experimental.pallas import tpu_sc as plsc

_NC = 2
_NS = 16
_NW = _NC * _NS
_CHUNK = 80
_NSLAB = 10


def _make_degree(n, e):
  ep = e // _NW
  nch = ep // _CHUNK
  slab = n // _NSLAB
  mesh = plsc.VectorSubcoreMesh(core_axis_name="c", subcore_axis_name="s")

  @functools.partial(
      pl.kernel,
      out_type=jax.ShapeDtypeStruct((_NC, n, 16), jnp.float32),
      mesh=mesh,
      scratch_types=[
          pltpu.VMEM((_CHUNK,), jnp.int32),
          pltpu.VMEM((_CHUNK, 16), jnp.float32),
          pltpu.VMEM_SHARED((n, 16), jnp.float32),
      ],
  )
  def deg_kernel(col_hbm, ones_hbm, zero_hbm, out_hbm, colbuf, onesbuf, acc):
    c = lax.axis_index("c")
    s = lax.axis_index("s")
    wid = s * _NC + c
    pltpu.sync_copy(ones_hbm, onesbuf)

    @pl.when(s < _NSLAB)
    def _():
      pltpu.sync_copy(zero_hbm.at[pl.ds(s * slab, slab)],
                      acc.at[pl.ds(s * slab, slab)])

    plsc.subcore_barrier()
    base = wid * ep

    @pl.loop(0, nch)
    def _(i):
      pltpu.sync_copy(col_hbm.at[pl.ds(base + i * _CHUNK, _CHUNK)], colbuf)
      pltpu.sync_copy(onesbuf, acc.at[colbuf], add=True)

    plsc.subcore_barrier()

    @pl.when(s < _NSLAB)
    def _():
      pltpu.sync_copy(acc.at[pl.ds(s * slab, slab)],
                      out_hbm.at[c, pl.ds(s * slab, slab)])

  return deg_kernel


def _make_scatter(n, e, d):
  ep = e // _NW
  nch = ep // _CHUNK
  slab = n // _NSLAB
  mesh = plsc.VectorSubcoreMesh(core_axis_name="c", subcore_axis_name="s")

  @functools.partial(
      pl.kernel,
      out_type=jax.ShapeDtypeStruct((_NC, n, d), jnp.float32),
      mesh=mesh,
      scratch_types=[
          pltpu.VMEM((_CHUNK,), jnp.int32),
          pltpu.VMEM((_CHUNK,), jnp.int32),
          pltpu.VMEM((_CHUNK, d), jnp.float32),
          pltpu.VMEM_SHARED((n, d), jnp.float32),
          pltpu.SemaphoreType.DMA,
      ],
  )
  def scat_kernel(vals_hbm, row_hbm, col_hbm, zero_hbm, out_hbm,
                  rowbuf, colbuf, valbuf, acc, sem):
    c = lax.axis_index("c")
    s = lax.axis_index("s")
    wid = s * _NC + c

    @pl.when(s < _NSLAB)
    def _():
      pltpu.sync_copy(zero_hbm.at[pl.ds(s * slab, slab)],
                      acc.at[pl.ds(s * slab, slab)])

    plsc.subcore_barrier()
    base = wid * ep

    @pl.loop(0, nch)
    def _(i):
      off = base + i * _CHUNK
      pltpu.sync_copy(row_hbm.at[pl.ds(off, _CHUNK)], rowbuf)
      pltpu.sync_copy(col_hbm.at[pl.ds(off, _CHUNK)], colbuf)
      pltpu.async_copy(vals_hbm.at[rowbuf], valbuf, sem).wait()
      pltpu.sync_copy(valbuf, acc.at[colbuf], add=True)

    plsc.subcore_barrier()

    @pl.when(s < _NSLAB)
    def _():
      pltpu.sync_copy(acc.at[pl.ds(s * slab, slab)],
                      out_hbm.at[c, pl.ds(s * slab, slab)])

  return scat_kernel


_BR = 1000


def _dinv_of(d_ref):
  deg = d_ref[0, :, 0] + d_ref[1, :, 0] + 1.0
  return lax.rsqrt(deg)[:, None]


def _tc1(x, w1, degp):
  n, in_dim = x.shape
  h = w1.shape[1]

  def body(x_ref, w_ref, d_ref, o_ref):
    hm = jnp.dot(x_ref[...], w_ref[...], preferred_element_type=jnp.float32)
    o_ref[...] = hm * _dinv_of(d_ref)

  return pl.pallas_call(
      body,
      grid=(n // _BR,),
      in_specs=[
          pl.BlockSpec((_BR, in_dim), lambda i: (i, 0)),
          pl.BlockSpec((in_dim, h), lambda i: (0, 0)),
          pl.BlockSpec((2, _BR, 16), lambda i: (0, i, 0)),
      ],
      out_specs=pl.BlockSpec((_BR, h), lambda i: (i, 0)),
      out_shape=jax.ShapeDtypeStruct((n, h), jnp.float32),
  )(x, w1, degp)


def _tc2(p, hs1, degp, w2, b1):
  n, h = hs1.shape
  h1 = w2.shape[1]

  def body(p_ref, hs_ref, d_ref, w_ref, b_ref, o_ref):
    dinv = _dinv_of(d_ref)
    t = (p_ref[0] + p_ref[1] + hs_ref[...]) * dinv + b_ref[0]
    r = jnp.maximum(t, 0.0)
    g = jnp.dot(r, w_ref[...], preferred_element_type=jnp.float32)
    o_ref[...] = g * dinv

  return pl.pallas_call(
      body,
      grid=(n // _BR,),
      in_specs=[
          pl.BlockSpec((2, _BR, h), lambda i: (0, i, 0)),
          pl.BlockSpec((_BR, h), lambda i: (i, 0)),
          pl.BlockSpec((2, _BR, 16), lambda i: (0, i, 0)),
          pl.BlockSpec((h, h1), lambda i: (0, 0)),
          pl.BlockSpec((1, h), lambda i: (0, 0)),
      ],
      out_specs=pl.BlockSpec((_BR, h1), lambda i: (i, 0)),
      out_shape=jax.ShapeDtypeStruct((n, h1), jnp.float32),
  )(p, hs1, degp, w2, b1)


def _tc3(q, gs2, degp, w3, b2, b3):
  n, h1 = gs2.shape
  out_d = w3.shape[1]

  def body(q_ref, g_ref, d_ref, w_ref, b2_ref, b3_ref, o_ref):
    dinv = _dinv_of(d_ref)
    t = (q_ref[0] + q_ref[1] + g_ref[...]) * dinv + b2_ref[0]
    o = jnp.dot(t, w_ref[...], preferred_element_type=jnp.float32) + b3_ref[0]
    m = jnp.max(o, axis=1, keepdims=True)
    sh = o - m
    lse = jnp.log(jnp.sum(jnp.exp(sh), axis=1, keepdims=True))
    o_ref[...] = sh - lse

  return pl.pallas_call(
      body,
      grid=(n // _BR,),
      in_specs=[
          pl.BlockSpec((2, _BR, h1), lambda i: (0, i, 0)),
          pl.BlockSpec((_BR, h1), lambda i: (i, 0)),
          pl.BlockSpec((2, _BR, 16), lambda i: (0, i, 0)),
          pl.BlockSpec((h1, out_d), lambda i: (0, 0)),
          pl.BlockSpec((1, h1), lambda i: (0, 0)),
          pl.BlockSpec((1, out_d), lambda i: (0, 0)),
      ],
      out_specs=pl.BlockSpec((_BR, out_d), lambda i: (i, 0)),
      out_shape=jax.ShapeDtypeStruct((n, out_d), jnp.float32),
  )(q, gs2, degp, w3, b2, b3)


@jax.jit
def _impl(x, edge_index, w1, b1, w2, b2, w3, b3):
  n = x.shape[0]
  e = edge_index.shape[1]
  h = w1.shape[1]
  h1 = w2.shape[1]
  row = edge_index[0]
  col = edge_index[1]

  ones16 = jnp.ones((_CHUNK, 16), jnp.float32)
  z16 = jnp.zeros((n, 16), jnp.float32)
  zh = jnp.zeros((n, h), jnp.float32)
  zh1 = jnp.zeros((n, h1), jnp.float32)

  degp = _make_degree(n, e)(col, ones16, z16)
  hs1 = _tc1(x, w1, degp)
  p = _make_scatter(n, e, h)(hs1, row, col, zh)
  gs2 = _tc2(p, hs1, degp, w2, b1.reshape(1, h))
  q = _make_scatter(n, e, h1)(gs2, row, col, zh1)
  return _tc3(q, gs2, degp, w3, b2.reshape(1, h1), b3.reshape(1, -1))


def kernel(x, edge_index, W1, b1, W2, b2, W3, b3):
  return _impl(x, edge_index, W1, b1, W2, b2, W3, b3)


# speedup vs baseline: 13.6081x; 13.6081x over previous
import functools

import jax
import jax.numpy as jnp
from jax import lax
from jax.experimental import pallas as pl
from jax.experimental.pallas import tpu as pltpu
from jax.experimental.pallas import tpu_sc as plsc

_NC = 2
_NS = 16
_NW = _NC * _NS
_CHUNK = 80
_NSLAB = 10

_SC_PARAMS = pltpu.CompilerParams(use_tc_tiling_on_sc=False)


def _make_degree(n, e):
  ep = e // _NW
  nch = ep // _CHUNK
  slab = n // _NSLAB
  mesh = plsc.VectorSubcoreMesh(core_axis_name="c", subcore_axis_name="s")

  @functools.partial(
      pl.kernel,
      out_type=jax.ShapeDtypeStruct((_NC, n, 16), jnp.float32),
      mesh=mesh,
      scratch_types=[
          pltpu.VMEM((_CHUNK,), jnp.int32),
          pltpu.VMEM((_CHUNK, 16), jnp.float32),
          pltpu.VMEM_SHARED((n, 16), jnp.float32),
      ],
      compiler_params=_SC_PARAMS,
  )
  def deg_kernel(col_hbm, ones_hbm, zero_hbm, out_hbm, colbuf, onesbuf, acc):
    c = lax.axis_index("c")
    s = lax.axis_index("s")
    wid = s * _NC + c
    pltpu.sync_copy(ones_hbm, onesbuf)

    @pl.when(s < _NSLAB)
    def _():
      pltpu.sync_copy(zero_hbm.at[pl.ds(s * slab, slab)],
                      acc.at[pl.ds(s * slab, slab)])

    plsc.subcore_barrier()
    base = wid * ep

    @pl.loop(0, nch)
    def _(i):
      pltpu.sync_copy(col_hbm.at[pl.ds(base + i * _CHUNK, _CHUNK)], colbuf)
      pltpu.sync_copy(onesbuf, acc.at[colbuf], add=True)

    plsc.subcore_barrier()

    @pl.when(s < _NSLAB)
    def _():
      pltpu.sync_copy(acc.at[pl.ds(s * slab, slab)],
                      out_hbm.at[c, pl.ds(s * slab, slab)])

  return deg_kernel


def _make_scatter(n, e, d):
  ep = e // _NW
  nch = ep // _CHUNK
  slab = n // _NSLAB
  mesh = plsc.VectorSubcoreMesh(core_axis_name="c", subcore_axis_name="s")

  @functools.partial(
      pl.kernel,
      out_type=jax.ShapeDtypeStruct((_NC, n, d), jnp.float32),
      mesh=mesh,
      scratch_types=[
          pltpu.VMEM((_CHUNK,), jnp.int32),
          pltpu.VMEM((_CHUNK,), jnp.int32),
          pltpu.VMEM((_CHUNK, d), jnp.float32),
          pltpu.VMEM_SHARED((n, d), jnp.float32),
          pltpu.SemaphoreType.DMA,
      ],
      compiler_params=_SC_PARAMS,
  )
  def scat_kernel(vals_hbm, row_hbm, col_hbm, zero_hbm, out_hbm,
                  rowbuf, colbuf, valbuf, acc, sem):
    c = lax.axis_index("c")
    s = lax.axis_index("s")
    wid = s * _NC + c

    @pl.when(s < _NSLAB)
    def _():
      pltpu.sync_copy(zero_hbm.at[pl.ds(s * slab, slab)],
                      acc.at[pl.ds(s * slab, slab)])

    plsc.subcore_barrier()
    base = wid * ep

    @pl.loop(0, nch)
    def _(i):
      off = base + i * _CHUNK
      pltpu.sync_copy(row_hbm.at[pl.ds(off, _CHUNK)], rowbuf)
      pltpu.sync_copy(col_hbm.at[pl.ds(off, _CHUNK)], colbuf)
      pltpu.async_copy(vals_hbm.at[rowbuf], valbuf, sem).wait()
      pltpu.sync_copy(valbuf, acc.at[colbuf], add=True)

    plsc.subcore_barrier()

    @pl.when(s < _NSLAB)
    def _():
      pltpu.sync_copy(acc.at[pl.ds(s * slab, slab)],
                      out_hbm.at[c, pl.ds(s * slab, slab)])

  return scat_kernel


_BR = 1000


def _dinv_of(d_ref):
  deg = d_ref[0, :, 0] + d_ref[1, :, 0] + 1.0
  return lax.rsqrt(deg)[:, None]


def _tc1(x, w1, degp):
  n, in_dim = x.shape
  h = w1.shape[1]

  def body(x_ref, w_ref, d_ref, o_ref):
    hm = jnp.dot(x_ref[...], w_ref[...], preferred_element_type=jnp.float32)
    o_ref[...] = hm * _dinv_of(d_ref)

  return pl.pallas_call(
      body,
      grid=(n // _BR,),
      in_specs=[
          pl.BlockSpec((_BR, in_dim), lambda i: (i, 0)),
          pl.BlockSpec((in_dim, h), lambda i: (0, 0)),
          pl.BlockSpec((2, _BR, 16), lambda i: (0, i, 0)),
      ],
      out_specs=pl.BlockSpec((_BR, h), lambda i: (i, 0)),
      out_shape=jax.ShapeDtypeStruct((n, h), jnp.float32),
  )(x, w1, degp)


def _tc2(p, hs1, degp, w2, b1):
  n, h = hs1.shape
  h1 = w2.shape[1]

  def body(p_ref, hs_ref, d_ref, w_ref, b_ref, o_ref):
    dinv = _dinv_of(d_ref)
    t = (p_ref[0] + p_ref[1] + hs_ref[...]) * dinv + b_ref[0]
    r = jnp.maximum(t, 0.0)
    g = jnp.dot(r, w_ref[...], preferred_element_type=jnp.float32)
    o_ref[...] = g * dinv

  return pl.pallas_call(
      body,
      grid=(n // _BR,),
      in_specs=[
          pl.BlockSpec((2, _BR, h), lambda i: (0, i, 0)),
          pl.BlockSpec((_BR, h), lambda i: (i, 0)),
          pl.BlockSpec((2, _BR, 16), lambda i: (0, i, 0)),
          pl.BlockSpec((h, h1), lambda i: (0, 0)),
          pl.BlockSpec((1, h), lambda i: (0, 0)),
      ],
      out_specs=pl.BlockSpec((_BR, h1), lambda i: (i, 0)),
      out_shape=jax.ShapeDtypeStruct((n, h1), jnp.float32),
  )(p, hs1, degp, w2, b1)


def _tc3(q, gs2, degp, w3, b2, b3):
  n, h1 = gs2.shape
  out_d = w3.shape[1]

  def body(q_ref, g_ref, d_ref, w_ref, b2_ref, b3_ref, o_ref):
    dinv = _dinv_of(d_ref)
    t = (q_ref[0] + q_ref[1] + g_ref[...]) * dinv + b2_ref[0]
    o = jnp.dot(t, w_ref[...], preferred_element_type=jnp.float32) + b3_ref[0]
    m = jnp.max(o, axis=1, keepdims=True)
    sh = o - m
    lse = jnp.log(jnp.sum(jnp.exp(sh), axis=1, keepdims=True))
    o_ref[...] = sh - lse

  return pl.pallas_call(
      body,
      grid=(n // _BR,),
      in_specs=[
          pl.BlockSpec((2, _BR, h1), lambda i: (0, i, 0)),
          pl.BlockSpec((_BR, h1), lambda i: (i, 0)),
          pl.BlockSpec((2, _BR, 16), lambda i: (0, i, 0)),
          pl.BlockSpec((h1, out_d), lambda i: (0, 0)),
          pl.BlockSpec((1, h1), lambda i: (0, 0)),
          pl.BlockSpec((1, out_d), lambda i: (0, 0)),
      ],
      out_specs=pl.BlockSpec((_BR, out_d), lambda i: (i, 0)),
      out_shape=jax.ShapeDtypeStruct((n, out_d), jnp.float32),
  )(q, gs2, degp, w3, b2, b3)


@jax.jit
def _impl(x, edge_index, w1, b1, w2, b2, w3, b3):
  n = x.shape[0]
  e = edge_index.shape[1]
  h = w1.shape[1]
  h1 = w2.shape[1]
  row = edge_index[0]
  col = edge_index[1]

  ones16 = jnp.ones((_CHUNK, 16), jnp.float32)
  z16 = jnp.zeros((n, 16), jnp.float32)
  zh = jnp.zeros((n, h), jnp.float32)
  zh1 = jnp.zeros((n, h1), jnp.float32)

  degp = _make_degree(n, e)(col, ones16, z16)
  hs1 = _tc1(x, w1, degp)
  p = _make_scatter(n, e, h)(hs1, row, col, zh)
  gs2 = _tc2(p, hs1, degp, w2, b1.reshape(1, h))
  q = _make_scatter(n, e, h1)(gs2, row, col, zh1)
  return _tc3(q, gs2, degp, w3, b2.reshape(1, h1), b3.reshape(1, -1))


def kernel(x, edge_index, W1, b1, W2, b2, W3, b3):
  return _impl(x, edge_index, W1, b1, W2, b2, W3, b3)
